# Initial kernel scaffold; baseline (speedup 1.0000x reference)
import jax
import jax.numpy as jnp
from jax.experimental import pallas as pl


def kernel(input, weight):
    raise NotImplementedError("write your pallas kernel here")


# speedup vs baseline: 1.7980x; 1.7980x over previous
import jax
import jax.numpy as jnp
from jax import lax
from jax.experimental import pallas as pl
from jax.experimental.pallas import tpu as pltpu
from jax.experimental.pallas import tpu_sc as plsc

_D = 64
_NC, _NS = 2, 16
_NW = _NC * _NS
_SUB = 128
_SPC = 4
_CHUNK = _SUB * _SPC


def _gather_body(idx_hbm, table_hbm, out_hbm, idx_v, rows_v, sem):
    wid = lax.axis_index("s") * _NC + lax.axis_index("c")
    subs_per_w = idx_hbm.shape[0] // _NW
    chunks_per_w = subs_per_w // _SPC

    def chunk(g, carry):
        sub_base = wid * subs_per_w + g * _SPC
        pltpu.sync_copy(idx_hbm.at[pl.ds(sub_base, _SPC)], idx_v)
        copies = [
            pltpu.async_copy(
                table_hbm.at[idx_v.at[j]],
                rows_v.at[pl.ds(j * _SUB, _SUB)],
                sem,
            )
            for j in range(_SPC)
        ]
        for c in copies:
            c.wait()
        pltpu.sync_copy(rows_v, out_hbm.at[pl.ds(sub_base * _SUB, _CHUNK)])
        return carry

    lax.fori_loop(0, chunks_per_w, chunk, 0)


def kernel(input, weight):
    b, h = input.shape
    rows = b * h
    idx = input.reshape(rows).astype(jnp.int32).reshape(rows // _SUB, _SUB)
    f = pl.kernel(
        _gather_body,
        out_type=jax.ShapeDtypeStruct((rows, _D), jnp.float32),
        mesh=plsc.VectorSubcoreMesh(core_axis_name="c", subcore_axis_name="s"),
        scratch_types=[
            pltpu.VMEM((_SPC, _SUB), jnp.int32),
            pltpu.VMEM((_CHUNK, _D), jnp.float32),
            pltpu.SemaphoreType.DMA,
        ],
        compiler_params=pltpu.CompilerParams(use_tc_tiling_on_sc=False),
    )
    out = f(idx, weight)
    return out.reshape(b, h, _D)


# speedup vs baseline: 1.8798x; 1.0454x over previous
import jax
import jax.numpy as jnp
from jax import lax
from jax.experimental import pallas as pl
from jax.experimental.pallas import tpu as pltpu
from jax.experimental.pallas import tpu_sc as plsc

_D = 64
_NC, _NS = 2, 16
_NW = _NC * _NS
_SUB = 128
_SPC = 2
_CHUNK = _SUB * _SPC
_NBUF = 6
_DR = 3
_PF = 3


def _gather_body(idx_hbm, table_hbm, out_hbm, idx_v, rows_v, *sems):
    isems, gsems, wsems = sems[:_NBUF], sems[_NBUF:2 * _NBUF], sems[2 * _NBUF:]
    wid = lax.axis_index("s") * _NC + lax.axis_index("c")
    subs_per_w = idx_hbm.shape[0] // _NW
    chunks_per_w = subs_per_w // _SPC
    sub0 = wid * subs_per_w

    def idx_copy(c, b, sem):
        return pltpu.make_async_copy(
            idx_hbm.at[pl.ds(sub0 + c * _SPC, _SPC)], idx_v.at[b], sem)

    def gather_copy(b, j, sem):
        return pltpu.make_async_copy(
            table_hbm.at[idx_v.at[b, j]],
            rows_v.at[b, pl.ds(j * _SUB, _SUB)], sem)

    def wb_copy(c, b, sem):
        return pltpu.make_async_copy(
            rows_v.at[b], out_hbm.at[pl.ds((sub0 + c * _SPC) * _SUB, _CHUNK)],
            sem)

    for c in range(_PF):
        idx_copy(c, c, isems[c]).start()

    def step(t, b, carry):
        c = t * _NBUF + b

        @pl.when(c < chunks_per_w)
        def _():
            @pl.when(c >= _NBUF)
            def _():
                wb_copy(0, b, wsems[b]).wait()
            idx_copy(0, b, isems[b]).wait()
            for j in range(_SPC):
                gather_copy(b, j, gsems[b]).start()

        k = c - _DR
        bp = (b - _DR) % _NBUF

        @pl.when((k >= 0) & (k < chunks_per_w))
        def _():
            for j in range(_SPC):
                gather_copy(bp, j, gsems[bp]).wait()
            wb_copy(k, bp, wsems[bp]).start()

        @pl.when(c + _PF < chunks_per_w)
        def _():
            bn = (b + _PF) % _NBUF
            idx_copy(c + _PF, bn, isems[bn]).start()
        return carry

    outer = -(-(chunks_per_w + _DR) // _NBUF)
    lax.fori_loop(
        0, outer,
        lambda t, cr: [step(t, b, cr) for b in range(_NBUF)][-1], 0)
    for b in range(_NBUF):
        wb_copy(0, b, wsems[b]).wait()


def kernel(input, weight):
    b, h = input.shape
    rows = b * h
    idx = input.reshape(rows).astype(jnp.int32).reshape(rows // _SUB, _SUB)
    f = pl.kernel(
        _gather_body,
        out_type=jax.ShapeDtypeStruct((rows, _D), jnp.float32),
        mesh=plsc.VectorSubcoreMesh(core_axis_name="c", subcore_axis_name="s"),
        scratch_types=(
            [pltpu.VMEM((_NBUF, _SPC, _SUB), jnp.int32),
             pltpu.VMEM((_NBUF, _CHUNK, _D), jnp.float32)]
            + [pltpu.SemaphoreType.DMA] * (3 * _NBUF)
        ),
        compiler_params=pltpu.CompilerParams(use_tc_tiling_on_sc=False),
    )
    out = f(idx, weight)
    return out.reshape(b, h, _D)
